# baseline (speedup 1.0000x reference)
import functools
import math

import numpy as np
import jax
import jax.numpy as jnp
from jax import lax
from jax.experimental import pallas as pl
from jax.experimental.pallas import tpu as pltpu
from jax.experimental.pallas import tpu_sc as plsc

BS_N = 2
D_MODEL_N = 256
H_N = 8
DH_N = 32
L_N = 4
P_N = 4
SHAPES_N = ((64, 64), (32, 32), (16, 16), (8, 8))
STARTS_N = (0, 4096, 5120, 5376)
LEN_N = 5440

QB1 = 544
QB3 = 272

def _col_consts():
    i = lax.broadcasted_iota(jnp.int32, (1, 128), 1)
    lvl = lax.rem(i // 4, 4)
    wi = jnp.right_shift(64, lvl)
    wf = wi.astype(jnp.float32)
    start = jnp.where(lvl == 0, 0,
                      jnp.where(lvl == 1, STARTS_N[1],
                                jnp.where(lvl == 2, STARTS_N[2],
                                          STARTS_N[3])))
    base0 = (i // 16) * LEN_N + start
    ei = lax.broadcasted_iota(jnp.int32, (4, 128), 0)
    ek = lax.rem(lax.broadcasted_iota(jnp.int32, (4, 128), 1) // 4, 4)
    esel = jnp.where(ei == ek, 1.0, 0.0).astype(jnp.float32)
    return wi, wf, base0, esel

SC_NC = 2
SC_NS = 16
SC_NW = SC_NC * SC_NS
TOT_ROWS = BS_N * H_N * LEN_N * 64
CHUNK = 128
NCHUNK = TOT_ROWS // CHUNK
CPT = NCHUNK // SC_NW
IBLK = 40
NBLK = CPT // IBLK
RING = 8
OPC = 16
TOT_OUT = NCHUNK * OPC


def _kv_proj_kernel(x_ref, wk_ref, bk_ref, wv_ref, bv_ref, kv_ref):
    x = x_ref[0]
    dn = (((1,), (1,)), ((), ()))
    kb = lax.dot_general(x, wk_ref[...], dn,
                         preferred_element_type=jnp.float32, precision=lax.Precision.HIGHEST) + bk_ref[...]
    vb = lax.dot_general(x, wv_ref[...], dn,
                         preferred_element_type=jnp.float32, precision=lax.Precision.HIGHEST) + bv_ref[...]
    for h in range(H_N):
        kv_ref[0, h, :, 0:DH_N] = kb[:, h * DH_N:(h + 1) * DH_N]
        kv_ref[0, h, :, DH_N:2 * DH_N] = vb[:, h * DH_N:(h + 1) * DH_N]


def _kv_proj(input_flatten, Wk, bk, Wv, bv):
    grid = (BS_N, LEN_N // QB1)
    return pl.pallas_call(
        _kv_proj_kernel,
        grid=grid,
        in_specs=[
            pl.BlockSpec((1, QB1, D_MODEL_N), lambda b, q: (b, q, 0)),
            pl.BlockSpec((D_MODEL_N, D_MODEL_N), lambda b, q: (0, 0)),
            pl.BlockSpec((1, D_MODEL_N), lambda b, q: (0, 0)),
            pl.BlockSpec((D_MODEL_N, D_MODEL_N), lambda b, q: (0, 0)),
            pl.BlockSpec((1, D_MODEL_N), lambda b, q: (0, 0)),
        ],
        out_specs=pl.BlockSpec((1, H_N, QB1, 2 * DH_N),
                               lambda b, q: (b, 0, q, 0)),
        out_shape=jax.ShapeDtypeStruct((BS_N, H_N, LEN_N, 2 * DH_N),
                                       jnp.float32),
    )(input_flatten, Wk, bk, Wv, bv)


def _qidx_kernel(x_ref, rx_ref, ry_ref, wq_ref, bq_ref, wox_ref, box_ref,
                 woy_ref, boy_ref, qh_ref, idx_ref, wt_ref):
    b = pl.program_id(0)
    x = x_ref[0]
    dn = (((1,), (1,)), ((), ()))
    q = lax.dot_general(x, wq_ref[...], dn,
                        preferred_element_type=jnp.float32, precision=lax.Precision.HIGHEST) + bq_ref[...]
    q = q * np.float32(1.0 / math.sqrt(DH_N))
    for h in range(H_N):
        qh_ref[0, h] = q[:, h * DH_N:(h + 1) * DH_N]

    offx = lax.dot_general(x, wox_ref[...], dn,
                           preferred_element_type=jnp.float32, precision=lax.Precision.HIGHEST) + box_ref[...]
    offy = lax.dot_general(x, woy_ref[...], dn,
                           preferred_element_type=jnp.float32, precision=lax.Precision.HIGHEST) + boy_ref[...]
    wi_c, wf_c, base0_c, esel_c = _col_consts()
    inv_c = 1.0 / wf_c
    dn2 = (((1,), (0,)), ((), ()))
    rxe = lax.dot_general(rx_ref[0], esel_c, dn2,
                          preferred_element_type=jnp.float32, precision=lax.Precision.HIGHEST)
    rye = lax.dot_general(ry_ref[0], esel_c, dn2,
                          preferred_element_type=jnp.float32, precision=lax.Precision.HIGHEST)

    px = (rxe + offx * inv_c) * wf_c - 0.5
    py = (rye + offy * inv_c) * wf_c - 0.5
    x0 = jnp.floor(px)
    y0 = jnp.floor(py)
    fx = px - x0
    fy = py - y0
    x1 = x0 + 1.0
    y1 = y0 + 1.0
    vx0 = (x0 >= 0.0) & (x0 < wf_c)
    vx1 = (x1 >= 0.0) & (x1 < wf_c)
    vy0 = (y0 >= 0.0) & (y0 < wf_c)
    vy1 = (y1 >= 0.0) & (y1 < wf_c)
    x0i = jnp.where(vx0, x0, 0.0).astype(jnp.int32)
    x1i = jnp.where(vx1, x1, 0.0).astype(jnp.int32)
    y0i = jnp.where(vy0, y0, 0.0).astype(jnp.int32)
    y1i = jnp.where(vy1, y1, 0.0).astype(jnp.int32)

    base = base0_c + b * (H_N * LEN_N)
    row0 = base + y0i * wi_c
    row1 = base + y1i * wi_c
    ia = row0 + x0i
    ib = row1 + x0i
    ic = row0 + x1i
    idd = row1 + x1i

    one = np.float32(1.0)
    ma = jnp.where(vx0 & vy0, one, 0.0)
    mb = jnp.where(vx0 & vy1, one, 0.0)
    mc = jnp.where(vx1 & vy0, one, 0.0)
    md = jnp.where(vx1 & vy1, one, 0.0)
    wa = (1.0 - fx) * (1.0 - fy) * ma
    wb = (1.0 - fx) * fy * mb
    wc = fx * (1.0 - fy) * mc
    wd = fx * fy * md

    for h in range(H_N):
        sl = slice(h * 16, h * 16 + 16)
        idx_ref[0, h, :, 0:16] = ia[:, sl]
        idx_ref[0, h, :, 16:32] = ib[:, sl]
        idx_ref[0, h, :, 32:48] = ic[:, sl]
        idx_ref[0, h, :, 48:64] = idd[:, sl]
        wt_ref[0, h, :, 0:16] = wa[:, sl]
        wt_ref[0, h, :, 16:32] = wb[:, sl]
        wt_ref[0, h, :, 32:48] = wc[:, sl]
        wt_ref[0, h, :, 48:64] = wd[:, sl]


def _qidx(query, rx, ry, Wq, bq, Woffx, boffx, Woffy, boffy):
    grid = (BS_N, LEN_N // QB1)
    return pl.pallas_call(
        _qidx_kernel,
        grid=grid,
        in_specs=[
            pl.BlockSpec((1, QB1, D_MODEL_N), lambda b, q: (b, q, 0)),
            pl.BlockSpec((1, QB1, L_N), lambda b, q: (b, q, 0)),
            pl.BlockSpec((1, QB1, L_N), lambda b, q: (b, q, 0)),
            pl.BlockSpec((D_MODEL_N, D_MODEL_N), lambda b, q: (0, 0)),
            pl.BlockSpec((1, D_MODEL_N), lambda b, q: (0, 0)),
            pl.BlockSpec((128, D_MODEL_N), lambda b, q: (0, 0)),
            pl.BlockSpec((1, 128), lambda b, q: (0, 0)),
            pl.BlockSpec((128, D_MODEL_N), lambda b, q: (0, 0)),
            pl.BlockSpec((1, 128), lambda b, q: (0, 0)),
        ],
        out_specs=[
            pl.BlockSpec((1, H_N, QB1, DH_N), lambda b, q: (b, 0, q, 0)),
            pl.BlockSpec((1, H_N, QB1, 64), lambda b, q: (b, 0, q, 0)),
            pl.BlockSpec((1, H_N, QB1, 64), lambda b, q: (b, 0, q, 0)),
        ],
        out_shape=[
            jax.ShapeDtypeStruct((BS_N, H_N, LEN_N, DH_N), jnp.float32),
            jax.ShapeDtypeStruct((BS_N, H_N, LEN_N, 64), jnp.int32),
            jax.ShapeDtypeStruct((BS_N, H_N, LEN_N, 64), jnp.float32),
        ],
    )(query, rx, ry, Wq, bq, Woffx, boffx, Woffy, boffy)


def _gather_body(kv_hbm, idx_hbm, wt_hbm, out_hbm, idx_v, wt_v, rows_v,
                 out_v, gsem, wsem):
    wid = lax.axis_index("s") * SC_NC + lax.axis_index("c")
    c0 = wid * CPT

    def wait_gather(r):
        pltpu.make_async_copy(
            kv_hbm.at[pl.ds(0, CHUNK)], rows_v.at[r], gsem.at[r]).wait()

    def wait_write(r):
        pltpu.make_async_copy(
            out_v.at[r], out_hbm.at[pl.ds(0, OPC)], wsem.at[r]).wait()

    def stage(bi):
        pb = lax.rem(bi, 2)
        pltpu.sync_copy(idx_hbm.at[pl.ds(c0 + bi * IBLK, IBLK)],
                        idx_v.at[pb])
        pltpu.sync_copy(wt_hbm.at[pl.ds(c0 + bi * IBLK, IBLK)],
                        wt_v.at[pb])

    def fire_gather(j):
        pbn = lax.rem(j // IBLK, 2)
        jjn = lax.rem(j, IBLK)
        r = lax.rem(j, RING)
        pltpu.make_async_copy(
            kv_hbm.at[idx_v.at[pbn, jjn]], rows_v.at[r], gsem.at[r]).start()

    stage(0)
    for j0 in range(RING):
        fire_gather(j0)

    lanes16 = [jnp.full((16,), lp, jnp.int32) for lp in range(16)]

    def blk_body(bi, carry):
        pb = lax.rem(bi, 2)

        @pl.when(bi + 1 < NBLK)
        def _():
            stage(bi + 1)

        def chunk_body(jj, carry2):
            j = bi * IBLK + jj
            r = lax.rem(j, RING)
            wait_gather(r)

            @pl.when(j >= RING)
            def _():
                wait_write(r)

            for qq in range(2):
                wvecs = [wt_v[pb, jj, pl.ds(qq * 64 + c * 16, 16)]
                         for c in range(4)]
                for lp in range(16):
                    wspl = [jnp.take(wvecs[c], lanes16[lp]) for c in range(4)]
                    orow = qq * 8 + lp // 2
                    ocol = (lp % 2) * 64
                    for s in range(4):
                        acc = wspl[0] * rows_v[r, qq * 64 + lp,
                                               pl.ds(s * 16, 16)]
                        acc += wspl[1] * rows_v[r, qq * 64 + 16 + lp,
                                                pl.ds(s * 16, 16)]
                        acc += wspl[2] * rows_v[r, qq * 64 + 32 + lp,
                                                pl.ds(s * 16, 16)]
                        acc += wspl[3] * rows_v[r, qq * 64 + 48 + lp,
                                                pl.ds(s * 16, 16)]
                        out_v[r, orow, pl.ds(ocol + s * 16, 16)] = acc

            pltpu.make_async_copy(
                out_v.at[r], out_hbm.at[pl.ds((c0 + j) * OPC, OPC)],
                wsem.at[r]).start()

            @pl.when(j + RING < CPT)
            def _():
                fire_gather(j + RING)

            return carry2

        lax.fori_loop(0, IBLK, chunk_body, 0)
        return carry

    lax.fori_loop(0, NBLK, blk_body, 0)
    for r in range(RING):
        wait_write(r)


def _gather_call(kv_flat, idx2d, wt2d):
    f = functools.partial(
        pl.kernel,
        out_type=jax.ShapeDtypeStruct((TOT_OUT, 128), jnp.float32),
        mesh=plsc.VectorSubcoreMesh(core_axis_name="c", subcore_axis_name="s"),
        scratch_types=[
            pltpu.VMEM((2, IBLK, CHUNK), jnp.int32),
            pltpu.VMEM((2, IBLK, CHUNK), jnp.float32),
            pltpu.VMEM((RING, CHUNK, 2 * DH_N), jnp.float32),
            pltpu.VMEM((RING, OPC, 128), jnp.float32),
            pltpu.SemaphoreType.DMA((RING,)),
            pltpu.SemaphoreType.DMA((RING,)),
        ],
        compiler_params=pltpu.CompilerParams(use_tc_tiling_on_sc=False),
    )(_gather_body)
    return f(kv_flat, idx2d, wt2d)


def _attn_kernel(g_ref, qh_ref, wot_ref, bo_ref, out_ref):
    h = pl.program_id(2)
    g = g_ref[0, 0]
    q = qh_ref[0, 0]
    hp = lax.Precision.HIGHEST
    cc = lax.broadcasted_iota(jnp.int32, (DH_N, 1024), 1)
    dd = lax.broadcasted_iota(jnp.int32, (DH_N, 1024), 0)
    tt = lax.rem(cc, 128)
    rep = jnp.where(lax.rem(tt, DH_N) == dd, 1.0, 0.0)
    dn = (((1,), (0,)), ((), ()))
    qtile = lax.dot_general(q, rep, dn, preferred_element_type=jnp.float32,
                            precision=hp)
    gq = g * qtile
    c1 = lax.broadcasted_iota(jnp.int32, (1024, 16), 0)
    j1 = lax.broadcasted_iota(jnp.int32, (1024, 16), 1)
    t1 = lax.rem(c1, 128)
    pt = 2 * (c1 // 128) + jnp.where(t1 >= 2 * DH_N, 1, 0)
    k1 = (t1 < DH_N) | ((t1 >= 2 * DH_N) & (t1 < 3 * DH_N))
    sel = jnp.where(k1 & (pt == j1), 1.0, 0.0)
    logits = lax.dot_general(gq, sel, dn, preferred_element_type=jnp.float32,
                             precision=hp)
    m = jnp.max(logits, axis=1, keepdims=True)
    e = jnp.exp(logits - m)
    attn = e / jnp.sum(e, axis=1, keepdims=True)
    c2 = lax.broadcasted_iota(jnp.int32, (16, 1024), 1)
    j2 = lax.broadcasted_iota(jnp.int32, (16, 1024), 0)
    t2 = lax.rem(c2, 128)
    pt2 = 2 * (c2 // 128) + jnp.where(t2 >= 2 * DH_N, 1, 0)
    k2 = (t2 < DH_N) | ((t2 >= 2 * DH_N) & (t2 < 3 * DH_N))
    exp_t = jnp.where((~k2) & (pt2 == j2), 1.0, 0.0)
    attn1024 = lax.dot_general(attn, exp_t, dn,
                               preferred_element_type=jnp.float32,
                               precision=hp)
    gv = g * attn1024
    c3 = lax.broadcasted_iota(jnp.int32, (1024, DH_N), 0)
    d3 = lax.broadcasted_iota(jnp.int32, (1024, DH_N), 1)
    t3 = lax.rem(c3, 128)
    isv3 = ((t3 >= DH_N) & (t3 < 2 * DH_N)) | (t3 >= 3 * DH_N)
    vsel = jnp.where(isv3 & (lax.rem(t3, DH_N) == d3), 1.0, 0.0)
    num = lax.dot_general(gv, vsel, dn, preferred_element_type=jnp.float32,
                          precision=hp)
    part = lax.dot_general(num, wot_ref[...], dn,
                           preferred_element_type=jnp.float32,
                           precision=hp)

    @pl.when(h == 0)
    def _():
        out_ref[0] = part + bo_ref[...]

    @pl.when(h != 0)
    def _():
        out_ref[0] = out_ref[0] + part


def _attn(g4, qh, WoT, bo):
    grid = (BS_N, LEN_N // QB3, H_N)
    return pl.pallas_call(
        _attn_kernel,
        grid=grid,
        in_specs=[
            pl.BlockSpec((1, 1, QB3, 1024), lambda b, qb, h: (b, h, qb, 0)),
            pl.BlockSpec((1, 1, QB3, DH_N), lambda b, qb, h: (b, h, qb, 0)),
            pl.BlockSpec((DH_N, D_MODEL_N), lambda b, qb, h: (h, 0)),
            pl.BlockSpec((1, D_MODEL_N), lambda b, qb, h: (0, 0)),
        ],
        out_specs=pl.BlockSpec((1, QB3, D_MODEL_N),
                               lambda b, qb, h: (b, qb, 0)),
        out_shape=jax.ShapeDtypeStruct((BS_N, LEN_N, D_MODEL_N), jnp.float32),
    )(g4, qh, WoT, bo)


def kernel(query, reference_points, input_flatten, input_spatial_shapes,
           input_level_start_index, Wq, bq, Wk, bk, Wv, bv, Wo, bo,
           Woff, boff):
    kv = _kv_proj(input_flatten, Wk, bk.reshape(1, -1), Wv,
                  bv.reshape(1, -1))
    rx = reference_points[..., 0]
    ry = reference_points[..., 1]
    qh, idx, wt = _qidx(query, rx, ry, Wq, bq.reshape(1, -1),
                        Woff[0::2], boff[0::2].reshape(1, -1),
                        Woff[1::2], boff[1::2].reshape(1, -1))
    g = _gather_call(kv.reshape(-1, 2 * DH_N), idx.reshape(NCHUNK, CHUNK),
                     wt.reshape(NCHUNK, CHUNK))
    g4 = g.reshape(BS_N, H_N, LEN_N, 1024)
    return _attn(g4, qh, jnp.transpose(Wo), bo.reshape(1, -1))


# speedup vs baseline: 1.2957x; 1.2957x over previous
import functools
import math

import numpy as np
import jax
import jax.numpy as jnp
from jax import lax
from jax.experimental import pallas as pl
from jax.experimental.pallas import tpu as pltpu
from jax.experimental.pallas import tpu_sc as plsc

BS_N = 2
D_MODEL_N = 256
H_N = 8
DH_N = 32
L_N = 4
P_N = 4
SHAPES_N = ((64, 64), (32, 32), (16, 16), (8, 8))
STARTS_N = (0, 4096, 5120, 5376)
LEN_N = 5440

QB1 = 544
QB3 = 272

def _col_consts():
    i = lax.broadcasted_iota(jnp.int32, (1, 128), 1)
    lvl = lax.rem(i // 4, 4)
    wi = jnp.right_shift(64, lvl)
    wf = wi.astype(jnp.float32)
    start = jnp.where(lvl == 0, 0,
                      jnp.where(lvl == 1, STARTS_N[1],
                                jnp.where(lvl == 2, STARTS_N[2],
                                          STARTS_N[3])))
    base0 = (i // 16) * LEN_N + start
    ei = lax.broadcasted_iota(jnp.int32, (4, 128), 0)
    ek = lax.rem(lax.broadcasted_iota(jnp.int32, (4, 128), 1) // 4, 4)
    esel = jnp.where(ei == ek, 1.0, 0.0).astype(jnp.float32)
    return wi, wf, base0, esel

SC_NC = 2
SC_NS = 16
SC_NW = SC_NC * SC_NS
TOT_ROWS = BS_N * H_N * LEN_N * 64
CHUNK = 128
NCHUNK = TOT_ROWS // CHUNK
CPT = NCHUNK // SC_NW
IBLK = 40
NBLK = CPT // IBLK
RING = 8
OPC = 16
TOT_OUT = NCHUNK * OPC


def _kv_proj_kernel(x_ref, wk_ref, bk_ref, wv_ref, bv_ref, kv_ref):
    x = x_ref[0]
    dn = (((1,), (1,)), ((), ()))
    kb = lax.dot_general(x, wk_ref[...], dn,
                         preferred_element_type=jnp.float32, precision=lax.Precision.HIGHEST) + bk_ref[...]
    vb = lax.dot_general(x, wv_ref[...], dn,
                         preferred_element_type=jnp.float32, precision=lax.Precision.HIGHEST) + bv_ref[...]
    for h in range(H_N):
        kv_ref[0, h, :, 0:DH_N] = kb[:, h * DH_N:(h + 1) * DH_N]
        kv_ref[0, h, :, DH_N:2 * DH_N] = vb[:, h * DH_N:(h + 1) * DH_N]


def _kv_proj(input_flatten, Wk, bk, Wv, bv):
    grid = (BS_N, LEN_N // QB1)
    return pl.pallas_call(
        _kv_proj_kernel,
        grid=grid,
        in_specs=[
            pl.BlockSpec((1, QB1, D_MODEL_N), lambda b, q: (b, q, 0)),
            pl.BlockSpec((D_MODEL_N, D_MODEL_N), lambda b, q: (0, 0)),
            pl.BlockSpec((1, D_MODEL_N), lambda b, q: (0, 0)),
            pl.BlockSpec((D_MODEL_N, D_MODEL_N), lambda b, q: (0, 0)),
            pl.BlockSpec((1, D_MODEL_N), lambda b, q: (0, 0)),
        ],
        out_specs=pl.BlockSpec((1, H_N, QB1, 2 * DH_N),
                               lambda b, q: (b, 0, q, 0)),
        out_shape=jax.ShapeDtypeStruct((BS_N, H_N, LEN_N, 2 * DH_N),
                                       jnp.float32),
    )(input_flatten, Wk, bk, Wv, bv)


def _qidx_kernel(x_ref, rx_ref, ry_ref, wq_ref, bq_ref, wox_ref, box_ref,
                 woy_ref, boy_ref, qh_ref, idx_ref, wt_ref):
    b = pl.program_id(0)
    x = x_ref[0]
    dn = (((1,), (1,)), ((), ()))
    q = lax.dot_general(x, wq_ref[...], dn,
                        preferred_element_type=jnp.float32, precision=lax.Precision.HIGHEST) + bq_ref[...]
    q = q * np.float32(1.0 / math.sqrt(DH_N))
    for h in range(H_N):
        qh_ref[0, h] = q[:, h * DH_N:(h + 1) * DH_N]

    offx = lax.dot_general(x, wox_ref[...], dn,
                           preferred_element_type=jnp.float32, precision=lax.Precision.HIGHEST) + box_ref[...]
    offy = lax.dot_general(x, woy_ref[...], dn,
                           preferred_element_type=jnp.float32, precision=lax.Precision.HIGHEST) + boy_ref[...]
    wi_c, wf_c, base0_c, esel_c = _col_consts()
    inv_c = 1.0 / wf_c
    dn2 = (((1,), (0,)), ((), ()))
    rxe = lax.dot_general(rx_ref[0], esel_c, dn2,
                          preferred_element_type=jnp.float32, precision=lax.Precision.HIGHEST)
    rye = lax.dot_general(ry_ref[0], esel_c, dn2,
                          preferred_element_type=jnp.float32, precision=lax.Precision.HIGHEST)

    px = (rxe + offx * inv_c) * wf_c - 0.5
    py = (rye + offy * inv_c) * wf_c - 0.5
    x0 = jnp.floor(px)
    y0 = jnp.floor(py)
    fx = px - x0
    fy = py - y0
    x1 = x0 + 1.0
    y1 = y0 + 1.0
    vx0 = (x0 >= 0.0) & (x0 < wf_c)
    vx1 = (x1 >= 0.0) & (x1 < wf_c)
    vy0 = (y0 >= 0.0) & (y0 < wf_c)
    vy1 = (y1 >= 0.0) & (y1 < wf_c)
    x0i = jnp.where(vx0, x0, 0.0).astype(jnp.int32)
    x1i = jnp.where(vx1, x1, 0.0).astype(jnp.int32)
    y0i = jnp.where(vy0, y0, 0.0).astype(jnp.int32)
    y1i = jnp.where(vy1, y1, 0.0).astype(jnp.int32)

    base = base0_c + b * (H_N * LEN_N)
    row0 = base + y0i * wi_c
    row1 = base + y1i * wi_c
    ia = row0 + x0i
    ib = row1 + x0i
    ic = row0 + x1i
    idd = row1 + x1i

    one = np.float32(1.0)
    ma = jnp.where(vx0 & vy0, one, 0.0)
    mb = jnp.where(vx0 & vy1, one, 0.0)
    mc = jnp.where(vx1 & vy0, one, 0.0)
    md = jnp.where(vx1 & vy1, one, 0.0)
    wa = (1.0 - fx) * (1.0 - fy) * ma
    wb = (1.0 - fx) * fy * mb
    wc = fx * (1.0 - fy) * mc
    wd = fx * fy * md

    for h in range(H_N):
        sl = slice(h * 16, h * 16 + 16)
        idx_ref[0, h, :, 0:16] = ia[:, sl]
        idx_ref[0, h, :, 16:32] = ib[:, sl]
        idx_ref[0, h, :, 32:48] = ic[:, sl]
        idx_ref[0, h, :, 48:64] = idd[:, sl]
        wt_ref[0, h, :, 0:16] = wa[:, sl]
        wt_ref[0, h, :, 16:32] = wb[:, sl]
        wt_ref[0, h, :, 32:48] = wc[:, sl]
        wt_ref[0, h, :, 48:64] = wd[:, sl]


def _qidx(query, rx, ry, Wq, bq, Woffx, boffx, Woffy, boffy):
    grid = (BS_N, LEN_N // QB1)
    return pl.pallas_call(
        _qidx_kernel,
        grid=grid,
        in_specs=[
            pl.BlockSpec((1, QB1, D_MODEL_N), lambda b, q: (b, q, 0)),
            pl.BlockSpec((1, QB1, L_N), lambda b, q: (b, q, 0)),
            pl.BlockSpec((1, QB1, L_N), lambda b, q: (b, q, 0)),
            pl.BlockSpec((D_MODEL_N, D_MODEL_N), lambda b, q: (0, 0)),
            pl.BlockSpec((1, D_MODEL_N), lambda b, q: (0, 0)),
            pl.BlockSpec((128, D_MODEL_N), lambda b, q: (0, 0)),
            pl.BlockSpec((1, 128), lambda b, q: (0, 0)),
            pl.BlockSpec((128, D_MODEL_N), lambda b, q: (0, 0)),
            pl.BlockSpec((1, 128), lambda b, q: (0, 0)),
        ],
        out_specs=[
            pl.BlockSpec((1, H_N, QB1, DH_N), lambda b, q: (b, 0, q, 0)),
            pl.BlockSpec((1, H_N, QB1, 64), lambda b, q: (b, 0, q, 0)),
            pl.BlockSpec((1, H_N, QB1, 64), lambda b, q: (b, 0, q, 0)),
        ],
        out_shape=[
            jax.ShapeDtypeStruct((BS_N, H_N, LEN_N, DH_N), jnp.float32),
            jax.ShapeDtypeStruct((BS_N, H_N, LEN_N, 64), jnp.int32),
            jax.ShapeDtypeStruct((BS_N, H_N, LEN_N, 64), jnp.float32),
        ],
    )(query, rx, ry, Wq, bq, Woffx, boffx, Woffy, boffy)


def _gather_body(kv_hbm, idx_hbm, wt_hbm, out_hbm, idx_v, wt_v, rows_v,
                 out_v, gsem, wsem):
    wid = lax.axis_index("s") * SC_NC + lax.axis_index("c")
    c0 = wid * CPT

    def wait_gather(r):
        pltpu.make_async_copy(
            kv_hbm.at[pl.ds(0, CHUNK)], rows_v.at[r], gsem.at[r]).wait()

    def wait_write(r):
        pltpu.make_async_copy(
            out_v.at[r], out_hbm.at[pl.ds(0, OPC)], wsem.at[r]).wait()

    def stage(bi):
        pb = lax.rem(bi, 2)
        pltpu.sync_copy(idx_hbm.at[pl.ds(c0 + bi * IBLK, IBLK)],
                        idx_v.at[pb])
        pltpu.sync_copy(wt_hbm.at[pl.ds(c0 + bi * IBLK, IBLK)],
                        wt_v.at[pb])

    def fire_gather(j):
        pbn = lax.rem(j // IBLK, 2)
        jjn = lax.rem(j, IBLK)
        r = lax.rem(j, RING)
        pltpu.make_async_copy(
            kv_hbm.at[idx_v.at[pbn, jjn]], rows_v.at[r], gsem.at[r]).start()

    stage(0)
    for j0 in range(RING):
        fire_gather(j0)

    lanes16 = [jnp.full((16,), lp, jnp.int32) for lp in range(16)]

    def blk_body(bi, carry):
        pb = lax.rem(bi, 2)

        @pl.when(bi + 1 < NBLK)
        def _():
            stage(bi + 1)

        def chunk_body(jj, carry2):
            j = bi * IBLK + jj
            r = lax.rem(j, RING)
            wait_gather(r)

            @pl.when(j >= RING)
            def _():
                wait_write(r)

            for qq in range(2):
                wvecs = [wt_v[pb, jj, pl.ds(qq * 64 + c * 16, 16)]
                         for c in range(4)]
                for lp in range(16):
                    wspl = [jnp.take(wvecs[c], lanes16[lp]) for c in range(4)]
                    orow = qq * 8 + lp // 2
                    ocol = (lp % 2) * 64
                    for s in range(4):
                        acc = wspl[0] * rows_v[r, qq * 64 + lp,
                                               pl.ds(s * 16, 16)]
                        acc += wspl[1] * rows_v[r, qq * 64 + 16 + lp,
                                                pl.ds(s * 16, 16)]
                        acc += wspl[2] * rows_v[r, qq * 64 + 32 + lp,
                                                pl.ds(s * 16, 16)]
                        acc += wspl[3] * rows_v[r, qq * 64 + 48 + lp,
                                                pl.ds(s * 16, 16)]
                        out_v[r, orow, pl.ds(ocol + s * 16, 16)] = acc

            pltpu.make_async_copy(
                out_v.at[r], out_hbm.at[pl.ds((c0 + j) * OPC, OPC)],
                wsem.at[r]).start()

            @pl.when(j + RING < CPT)
            def _():
                fire_gather(j + RING)

            return carry2

        lax.fori_loop(0, IBLK, chunk_body, 0)
        return carry

    lax.fori_loop(0, NBLK, blk_body, 0)
    for r in range(RING):
        wait_write(r)


def _gather_call(kv_flat, idx2d, wt2d):
    f = functools.partial(
        pl.kernel,
        out_type=jax.ShapeDtypeStruct((TOT_OUT, 128), jnp.float32),
        mesh=plsc.VectorSubcoreMesh(core_axis_name="c", subcore_axis_name="s"),
        scratch_types=[
            pltpu.VMEM((2, IBLK, CHUNK), jnp.int32),
            pltpu.VMEM((2, IBLK, CHUNK), jnp.float32),
            pltpu.VMEM((RING, CHUNK, 2 * DH_N), jnp.float32),
            pltpu.VMEM((RING, OPC, 128), jnp.float32),
            pltpu.SemaphoreType.DMA((RING,)),
            pltpu.SemaphoreType.DMA((RING,)),
        ],
        compiler_params=pltpu.CompilerParams(use_tc_tiling_on_sc=False),
    )(_gather_body)
    return f(kv_flat, idx2d, wt2d)


def _attn_kernel(g_ref, qh_ref, wot_ref, bo_ref, sel_ref, dg_ref, out_ref):
    h = pl.program_id(2)
    g = g_ref[0, 0]
    q = qh_ref[0, 0]
    z = jnp.zeros_like(q)
    qfull = jnp.concatenate([q, z, q, z], axis=1)
    gq = g * qfull[:, None, :]
    gq2 = gq.reshape(QB3 * 8, 128)
    lg2 = lax.dot_general(gq2, sel_ref[...], (((1,), (0,)), ((), ())),
                          preferred_element_type=jnp.float32,
                          precision=lax.Precision.HIGHEST)
    lg = lg2.reshape(QB3, 8, 16)
    logits = jnp.sum(lg * dg_ref[...][None], axis=1)
    m = jnp.max(logits, axis=1, keepdims=True)
    e = jnp.exp(logits - m)
    s = jnp.sum(e, axis=1, keepdims=True)
    ee = e[:, 0:8]
    eo = e[:, 8:16]
    zz = jnp.zeros((QB3, 8, DH_N), jnp.float32)
    afull = jnp.concatenate(
        [zz, jnp.broadcast_to(ee[:, :, None], (QB3, 8, DH_N)),
         zz, jnp.broadcast_to(eo[:, :, None], (QB3, 8, DH_N))], axis=2)
    gvs = jnp.sum(g * afull, axis=1)
    num = gvs[:, DH_N:2 * DH_N] + gvs[:, 3 * DH_N:4 * DH_N]
    outh = num / s
    part = lax.dot_general(outh, wot_ref[...], (((1,), (0,)), ((), ())),
                           preferred_element_type=jnp.float32,
                           precision=lax.Precision.HIGHEST)

    @pl.when(h == 0)
    def _():
        out_ref[0] = part + bo_ref[...]

    @pl.when(h != 0)
    def _():
        out_ref[0] = out_ref[0] + part


def _attn(g5, qh, WoT, bo):
    ci = np.arange(128)[:, None]
    ri = np.arange(16)[None, :]
    sel = (((ri < 8) & (ci < DH_N))
           | ((ri >= 8) & (ci >= 2 * DH_N) & (ci < 3 * DH_N))
           ).astype(np.float32)
    ii = np.arange(8)[:, None]
    jj = np.arange(16)[None, :]
    diag = ((jj % 8) == ii).astype(np.float32)
    grid = (BS_N, LEN_N // QB3, H_N)
    return pl.pallas_call(
        _attn_kernel,
        grid=grid,
        in_specs=[
            pl.BlockSpec((1, 1, QB3, 8, 128),
                         lambda b, qb, h: (b, h, qb, 0, 0)),
            pl.BlockSpec((1, 1, QB3, DH_N), lambda b, qb, h: (b, h, qb, 0)),
            pl.BlockSpec((DH_N, D_MODEL_N), lambda b, qb, h: (h, 0)),
            pl.BlockSpec((1, D_MODEL_N), lambda b, qb, h: (0, 0)),
            pl.BlockSpec((128, 16), lambda b, qb, h: (0, 0)),
            pl.BlockSpec((8, 16), lambda b, qb, h: (0, 0)),
        ],
        out_specs=pl.BlockSpec((1, QB3, D_MODEL_N),
                               lambda b, qb, h: (b, qb, 0)),
        out_shape=jax.ShapeDtypeStruct((BS_N, LEN_N, D_MODEL_N), jnp.float32),
    )(g5, qh, WoT, bo, jnp.asarray(sel), jnp.asarray(diag))


def kernel(query, reference_points, input_flatten, input_spatial_shapes,
           input_level_start_index, Wq, bq, Wk, bk, Wv, bv, Wo, bo,
           Woff, boff):
    kv = _kv_proj(input_flatten, Wk, bk.reshape(1, -1), Wv,
                  bv.reshape(1, -1))
    rx = reference_points[..., 0]
    ry = reference_points[..., 1]
    qh, idx, wt = _qidx(query, rx, ry, Wq, bq.reshape(1, -1),
                        Woff[0::2], boff[0::2].reshape(1, -1),
                        Woff[1::2], boff[1::2].reshape(1, -1))
    g = _gather_call(kv.reshape(-1, 2 * DH_N), idx.reshape(NCHUNK, CHUNK),
                     wt.reshape(NCHUNK, CHUNK))
    g5 = g.reshape(BS_N, H_N, LEN_N, 8, 128)
    return _attn(g5, qh, jnp.transpose(Wo), bo.reshape(1, -1))
